# Initial kernel scaffold; baseline (speedup 1.0000x reference)
import functools

import jax
import jax.numpy as jnp
from jax import lax
from jax.experimental import pallas as pl
from jax.experimental.pallas import tpu as pltpu
from jax.experimental.pallas import tpu_sc as plsc

N = 10000
NP = 10240
E = 320000
HID = 128
GRID = 4
BLK = 512
NBLK = NP // BLK

NSC = 2
NTILE = 16
NW = NSC * NTILE
EPW = E // NW
CH = 125
NCH = EPW // CH
RPT = NP // NTILE

_F32 = jnp.float32


def _kan1_body(x_ref, w_ref, w0_ref, o_ref):
    x = x_ref[...]
    x2 = x * x
    x3 = x2 * x
    x4 = x2 * x2
    acc = jnp.dot(x, w_ref[0], preferred_element_type=_F32)
    acc = acc + jnp.dot(x2, w_ref[1], preferred_element_type=_F32)
    acc = acc + jnp.dot(x3, w_ref[2], preferred_element_type=_F32)
    acc = acc + jnp.dot(x4, w_ref[3], preferred_element_type=_F32)
    o_ref[...] = acc + w0_ref[0:1, :]


def _kan1(hp, w, w0s):
    return pl.pallas_call(
        _kan1_body,
        grid=(NBLK,),
        in_specs=[
            pl.BlockSpec((BLK, HID), lambda i: (i, 0)),
            pl.BlockSpec((4, HID, HID), lambda i: (0, 0, 0)),
            pl.BlockSpec((8, HID), lambda i: (0, 0)),
        ],
        out_specs=pl.BlockSpec((BLK, HID), lambda i: (i, 0)),
        out_shape=jax.ShapeDtypeStruct((NP, HID), _F32),
    )(hp, w, w0s)


def _agg_body(h_hbm, src_hbm, dst_hbm, zero_hbm, out_hbm,
              srcv, dstv, rows0, rows1, acc, semg0, semg1):
    c = lax.axis_index("c")
    s = lax.axis_index("s")
    wid = c * NTILE + s

    pltpu.sync_copy(zero_hbm, acc.at[pl.ds(s * RPT, RPT)])

    pltpu.sync_copy(src_hbm.at[pl.ds(wid * NCH, NCH)], srcv)
    pltpu.sync_copy(dst_hbm.at[pl.ds(wid * NCH, NCH)], dstv)

    plsc.subcore_barrier()

    @pl.loop(0, NCH, step=2)
    def _(i):
        g0 = pltpu.async_copy(h_hbm.at[srcv.at[i]], rows0, semg0)
        g1 = pltpu.async_copy(h_hbm.at[srcv.at[i + 1]], rows1, semg1)
        g0.wait()
        pltpu.sync_copy(rows0, acc.at[dstv.at[i]], add=True)
        g1.wait()
        pltpu.sync_copy(rows1, acc.at[dstv.at[i + 1]], add=True)

    plsc.subcore_barrier()

    pltpu.sync_copy(acc.at[pl.ds(s * RPT, RPT)],
                    out_hbm.at[c].at[pl.ds(s * RPT, RPT)])


def _agg(hp, src2, dst2, zeros):
    mesh = plsc.VectorSubcoreMesh(core_axis_name="c", subcore_axis_name="s")
    kern = pl.kernel(
        _agg_body,
        out_type=jax.ShapeDtypeStruct((NSC, NP, HID), _F32),
        mesh=mesh,
        scratch_types=[
            pltpu.VMEM((NCH, CH), jnp.int32),
            pltpu.VMEM((NCH, CH), jnp.int32),
            pltpu.VMEM((CH, HID), _F32),
            pltpu.VMEM((CH, HID), _F32),
            pltpu.VMEM_SHARED((NP, HID), _F32),
            pltpu.SemaphoreType.DMA,
            pltpu.SemaphoreType.DMA,
        ],
    )
    return kern(hp, src2, dst2, zeros)


def _fourier_block(pa_ref, pb_ref, y_ref, wc_ref, ws_ref):
    a = pa_ref[0] + pb_ref[0]
    c1 = jnp.cos(a)
    s1 = jnp.sin(a)
    c2 = c1 * c1 - s1 * s1
    s2 = 2.0 * c1 * s1
    c3 = c2 * c1 - s2 * s1
    s3 = s2 * c1 + c2 * s1
    c4 = c2 * c2 - s2 * s2
    s4 = 2.0 * s2 * c2
    acc = jnp.dot(c1, wc_ref[0], preferred_element_type=_F32)
    acc = acc + jnp.dot(c2, wc_ref[1], preferred_element_type=_F32)
    acc = acc + jnp.dot(c3, wc_ref[2], preferred_element_type=_F32)
    acc = acc + jnp.dot(c4, wc_ref[3], preferred_element_type=_F32)
    acc = acc + jnp.dot(s1, ws_ref[0], preferred_element_type=_F32)
    acc = acc + jnp.dot(s2, ws_ref[1], preferred_element_type=_F32)
    acc = acc + jnp.dot(s3, ws_ref[2], preferred_element_type=_F32)
    acc = acc + jnp.dot(s4, ws_ref[3], preferred_element_type=_F32)
    r = acc + y_ref[...]
    return jnp.where(r >= 0.0, r, 0.01 * r)


def _mid_body(pa_ref, pb_ref, y_ref, wc_ref, ws_ref, o_ref):
    o_ref[...] = _fourier_block(pa_ref, pb_ref, y_ref, wc_ref, ws_ref)


def _mid(p, y1, wc, ws):
    return pl.pallas_call(
        _mid_body,
        grid=(NBLK,),
        in_specs=[
            pl.BlockSpec((1, BLK, HID), lambda i: (0, i, 0)),
            pl.BlockSpec((1, BLK, HID), lambda i: (1, i, 0)),
            pl.BlockSpec((BLK, HID), lambda i: (i, 0)),
            pl.BlockSpec((4, HID, HID), lambda i: (0, 0, 0)),
            pl.BlockSpec((4, HID, HID), lambda i: (0, 0, 0)),
        ],
        out_specs=pl.BlockSpec((BLK, HID), lambda i: (i, 0)),
        out_shape=jax.ShapeDtypeStruct((NP, HID), _F32),
    )(p, p, y1, wc, ws)


def _final_body(pa_ref, pb_ref, y_ref, wc_ref, ws_ref, lwb_ref, o_ref,
                pool_ref):
    i = pl.program_id(0)

    @pl.when(i == 0)
    def _():
        pool_ref[...] = jnp.zeros_like(pool_ref)

    r = _fourier_block(pa_ref, pb_ref, y_ref, wc_ref, ws_ref)
    rows = i * BLK + lax.broadcasted_iota(jnp.int32, (BLK, HID), 0)
    r = jnp.where(rows < N, r, 0.0)
    pool_ref[0:1, :] = pool_ref[0:1, :] + jnp.sum(r, axis=0, keepdims=True)

    @pl.when(i == NBLK - 1)
    def _():
        pool = pool_ref[0:1, :]
        z = jnp.sum(lwb_ref[0:1, :] + pool * lwb_ref[1:2, :])
        o_ref[...] = jax.nn.sigmoid(z).reshape(1, 1)


def _final(p, h2, wc, ws, lwb):
    return pl.pallas_call(
        _final_body,
        grid=(NBLK,),
        in_specs=[
            pl.BlockSpec((1, BLK, HID), lambda i: (0, i, 0)),
            pl.BlockSpec((1, BLK, HID), lambda i: (1, i, 0)),
            pl.BlockSpec((BLK, HID), lambda i: (i, 0)),
            pl.BlockSpec((4, HID, HID), lambda i: (0, 0, 0)),
            pl.BlockSpec((4, HID, HID), lambda i: (0, 0, 0)),
            pl.BlockSpec((8, HID), lambda i: (0, 0)),
        ],
        out_specs=pl.BlockSpec((1, 1), lambda i: (0, 0)),
        out_shape=jax.ShapeDtypeStruct((1, 1), _F32),
        scratch_shapes=[pltpu.VMEM((8, HID), _F32)],
    )(p, p, h2, wc, ws, lwb)


def kernel(h, edge_index, coeffs1, fc1, fc2, lin_coeffs, lin_bias):
    hp = jnp.pad(h, ((0, NP - N), (0, 0)))
    src2 = edge_index[0].reshape(E // CH, CH)
    dst2 = edge_index[1].reshape(E // CH, CH)
    zeros = jnp.zeros((RPT, HID), _F32)

    w = jnp.stack([coeffs1[:, :, d].T for d in (1, 2, 3, 4)])
    w0s = jnp.broadcast_to(jnp.sum(coeffs1[:, :, 0], axis=1)[None, :],
                           (8, HID))
    wc1 = jnp.transpose(fc1[0], (2, 1, 0))
    ws1 = jnp.transpose(fc1[1], (2, 1, 0))
    wc2 = jnp.transpose(fc2[0], (2, 1, 0))
    ws2 = jnp.transpose(fc2[1], (2, 1, 0))
    lw0 = lin_coeffs[0, :, 0] + lin_bias[0, 0] / HID
    lw1 = lin_coeffs[0, :, 1]
    lwb = jnp.concatenate(
        [lw0[None, :], lw1[None, :], jnp.zeros((6, HID), _F32)], axis=0)

    y1 = _kan1(hp, w, w0s)
    p1 = _agg(y1, src2, dst2, zeros)
    h2 = _mid(p1, y1, wc1, ws1)
    p2 = _agg(h2, src2, dst2, zeros)
    return _final(p2, h2, wc2, ws2, lwb)


# speedup vs baseline: 8.8742x; 8.8742x over previous
import jax
import jax.numpy as jnp
from jax import lax
from jax.experimental import pallas as pl
from jax.experimental.pallas import tpu as pltpu
from jax.experimental.pallas import tpu_sc as plsc

N = 10000
NP = 10240
E = 320000
HID = 128
HH = HID // 2
BLK = 512
NBLK = NP // BLK

NSC = 2
NTILE = 16
EPT = E // NTILE
CH = 125
NCH = EPT // CH
RPT = NP // NTILE

_F32 = jnp.float32


def _kan1_body(x_ref, w_ref, w0_ref, o_ref):
    x = x_ref[...]
    x2 = x * x
    x3 = x2 * x
    x4 = x2 * x2
    acc = jnp.dot(x, w_ref[0], preferred_element_type=_F32)
    acc = acc + jnp.dot(x2, w_ref[1], preferred_element_type=_F32)
    acc = acc + jnp.dot(x3, w_ref[2], preferred_element_type=_F32)
    acc = acc + jnp.dot(x4, w_ref[3], preferred_element_type=_F32)
    acc = acc + w0_ref[0:1, :]
    o_ref[0] = acc[:, :HH]
    o_ref[1] = acc[:, HH:]


def _kan1(hp, w, w0s):
    return pl.pallas_call(
        _kan1_body,
        grid=(NBLK,),
        in_specs=[
            pl.BlockSpec((BLK, HID), lambda i: (i, 0)),
            pl.BlockSpec((4, HID, HID), lambda i: (0, 0, 0)),
            pl.BlockSpec((8, HID), lambda i: (0, 0)),
        ],
        out_specs=pl.BlockSpec((2, BLK, HH), lambda i: (0, i, 0)),
        out_shape=jax.ShapeDtypeStruct((2, NP, HH), _F32),
    )(hp, w, w0s)


def _agg_body(h_hbm, src_hbm, dst_hbm, zero_hbm, out_hbm,
              srcv, dstv, rows0, rows1, acc, semg0, semg1):
    c = lax.axis_index("c")
    s = lax.axis_index("s")

    pltpu.sync_copy(zero_hbm, acc.at[pl.ds(s * RPT, RPT)])

    pltpu.sync_copy(src_hbm.at[pl.ds(s * NCH, NCH)], srcv)
    pltpu.sync_copy(dst_hbm.at[pl.ds(s * NCH, NCH)], dstv)

    plsc.subcore_barrier()

    hc = h_hbm.at[c]

    @pl.loop(0, NCH, step=2)
    def _(i):
        g0 = pltpu.async_copy(hc.at[srcv.at[i]], rows0, semg0)
        g1 = pltpu.async_copy(hc.at[srcv.at[i + 1]], rows1, semg1)
        g0.wait()
        pltpu.sync_copy(rows0, acc.at[dstv.at[i]], add=True)
        g1.wait()
        pltpu.sync_copy(rows1, acc.at[dstv.at[i + 1]], add=True)

    plsc.subcore_barrier()

    pltpu.sync_copy(acc.at[pl.ds(s * RPT, RPT)],
                    out_hbm.at[c].at[pl.ds(s * RPT, RPT)])


def _agg(hsplit, src2, dst2, zeros):
    mesh = plsc.VectorSubcoreMesh(core_axis_name="c", subcore_axis_name="s")
    kern = pl.kernel(
        _agg_body,
        out_type=jax.ShapeDtypeStruct((NSC, NP, HH), _F32),
        mesh=mesh,
        compiler_params=pltpu.CompilerParams(use_tc_tiling_on_sc=False),
        scratch_types=[
            pltpu.VMEM((NCH, CH), jnp.int32),
            pltpu.VMEM((NCH, CH), jnp.int32),
            pltpu.VMEM((CH, HH), _F32),
            pltpu.VMEM((CH, HH), _F32),
            pltpu.VMEM_SHARED((NP, HH), _F32),
            pltpu.SemaphoreType.DMA,
            pltpu.SemaphoreType.DMA,
        ],
    )
    return kern(hsplit, src2, dst2, zeros)


def _fourier_block(pa_ref, pb_ref, ya_ref, yb_ref, wc_ref, ws_ref):
    a = jnp.concatenate([pa_ref[0], pb_ref[0]], axis=1)
    c1 = jnp.cos(a)
    s1 = jnp.sin(a)
    c2 = c1 * c1 - s1 * s1
    s2 = 2.0 * c1 * s1
    c3 = c2 * c1 - s2 * s1
    s3 = s2 * c1 + c2 * s1
    c4 = c2 * c2 - s2 * s2
    s4 = 2.0 * s2 * c2
    acc = jnp.dot(c1, wc_ref[0], preferred_element_type=_F32)
    acc = acc + jnp.dot(c2, wc_ref[1], preferred_element_type=_F32)
    acc = acc + jnp.dot(c3, wc_ref[2], preferred_element_type=_F32)
    acc = acc + jnp.dot(c4, wc_ref[3], preferred_element_type=_F32)
    acc = acc + jnp.dot(s1, ws_ref[0], preferred_element_type=_F32)
    acc = acc + jnp.dot(s2, ws_ref[1], preferred_element_type=_F32)
    acc = acc + jnp.dot(s3, ws_ref[2], preferred_element_type=_F32)
    acc = acc + jnp.dot(s4, ws_ref[3], preferred_element_type=_F32)
    r = acc + jnp.concatenate([ya_ref[0], yb_ref[0]], axis=1)
    return jnp.where(r >= 0.0, r, 0.01 * r)


def _mid_body(pa_ref, pb_ref, ya_ref, yb_ref, wc_ref, ws_ref, o_ref):
    r = _fourier_block(pa_ref, pb_ref, ya_ref, yb_ref, wc_ref, ws_ref)
    o_ref[0] = r[:, :HH]
    o_ref[1] = r[:, HH:]


def _mid(p, y1, wc, ws):
    return pl.pallas_call(
        _mid_body,
        grid=(NBLK,),
        in_specs=[
            pl.BlockSpec((1, BLK, HH), lambda i: (0, i, 0)),
            pl.BlockSpec((1, BLK, HH), lambda i: (1, i, 0)),
            pl.BlockSpec((1, BLK, HH), lambda i: (0, i, 0)),
            pl.BlockSpec((1, BLK, HH), lambda i: (1, i, 0)),
            pl.BlockSpec((4, HID, HID), lambda i: (0, 0, 0)),
            pl.BlockSpec((4, HID, HID), lambda i: (0, 0, 0)),
        ],
        out_specs=pl.BlockSpec((2, BLK, HH), lambda i: (0, i, 0)),
        out_shape=jax.ShapeDtypeStruct((2, NP, HH), _F32),
    )(p, p, y1, y1, wc, ws)


def _final_body(pa_ref, pb_ref, ya_ref, yb_ref, wc_ref, ws_ref, lwb_ref,
                o_ref, pool_ref):
    i = pl.program_id(0)

    @pl.when(i == 0)
    def _():
        pool_ref[...] = jnp.zeros_like(pool_ref)

    r = _fourier_block(pa_ref, pb_ref, ya_ref, yb_ref, wc_ref, ws_ref)
    rows = i * BLK + lax.broadcasted_iota(jnp.int32, (BLK, HID), 0)
    r = jnp.where(rows < N, r, 0.0)
    pool_ref[0:1, :] = pool_ref[0:1, :] + jnp.sum(r, axis=0, keepdims=True)

    @pl.when(i == NBLK - 1)
    def _():
        pool = pool_ref[0:1, :]
        z = jnp.sum(lwb_ref[0:1, :] + pool * lwb_ref[1:2, :])
        o_ref[...] = jax.nn.sigmoid(z).reshape(1, 1)


def _final(p, h2, wc, ws, lwb):
    return pl.pallas_call(
        _final_body,
        grid=(NBLK,),
        in_specs=[
            pl.BlockSpec((1, BLK, HH), lambda i: (0, i, 0)),
            pl.BlockSpec((1, BLK, HH), lambda i: (1, i, 0)),
            pl.BlockSpec((1, BLK, HH), lambda i: (0, i, 0)),
            pl.BlockSpec((1, BLK, HH), lambda i: (1, i, 0)),
            pl.BlockSpec((4, HID, HID), lambda i: (0, 0, 0)),
            pl.BlockSpec((4, HID, HID), lambda i: (0, 0, 0)),
            pl.BlockSpec((8, HID), lambda i: (0, 0)),
        ],
        out_specs=pl.BlockSpec((1, 1), lambda i: (0, 0)),
        out_shape=jax.ShapeDtypeStruct((1, 1), _F32),
        scratch_shapes=[pltpu.VMEM((8, HID), _F32)],
    )(p, p, h2, h2, wc, ws, lwb)


def kernel(h, edge_index, coeffs1, fc1, fc2, lin_coeffs, lin_bias):
    hp = jnp.pad(h, ((0, NP - N), (0, 0)))
    src2 = edge_index[0].reshape(E // CH, CH)
    dst2 = edge_index[1].reshape(E // CH, CH)
    zeros = jnp.zeros((RPT, HH), _F32)

    w = jnp.stack([coeffs1[:, :, d].T for d in (1, 2, 3, 4)])
    w0s = jnp.broadcast_to(jnp.sum(coeffs1[:, :, 0], axis=1)[None, :],
                           (8, HID))
    wc1 = jnp.transpose(fc1[0], (2, 1, 0))
    ws1 = jnp.transpose(fc1[1], (2, 1, 0))
    wc2 = jnp.transpose(fc2[0], (2, 1, 0))
    ws2 = jnp.transpose(fc2[1], (2, 1, 0))
    lw0 = lin_coeffs[0, :, 0] + lin_bias[0, 0] / HID
    lw1 = lin_coeffs[0, :, 1]
    lwb = jnp.concatenate(
        [lw0[None, :], lw1[None, :], jnp.zeros((6, HID), _F32)], axis=0)

    y1 = _kan1(hp, w, w0s)
    p1 = _agg(y1, src2, dst2, zeros)
    h2 = _mid(p1, y1, wc1, ws1)
    p2 = _agg(h2, src2, dst2, zeros)
    return _final(p2, h2, wc2, ws2, lwb)


# speedup vs baseline: 10.1351x; 1.1421x over previous
import jax
import jax.numpy as jnp
from jax import lax
from jax.experimental import pallas as pl
from jax.experimental.pallas import tpu as pltpu
from jax.experimental.pallas import tpu_sc as plsc

N = 10000
NP = 10240
E = 320000
HID = 128
BLK = 512
NBLK = NP // BLK

NSC = 2
NTILE = 16
EPT = E // NTILE
HALF = NP // 2
TRASH = 128
ACC_R = HALF + TRASH
RPT = ACC_R // NTILE
CH = 128
CAPC = 158
CAP = CAPC * CH

_F32 = jnp.float32
_I32 = jnp.int32


def _kan1_body(x_ref, w_ref, w0_ref, o_ref):
    x = x_ref[...]
    x2 = x * x
    x3 = x2 * x
    x4 = x2 * x2
    acc = jnp.dot(x, w_ref[0], preferred_element_type=_F32)
    acc = acc + jnp.dot(x2, w_ref[1], preferred_element_type=_F32)
    acc = acc + jnp.dot(x3, w_ref[2], preferred_element_type=_F32)
    acc = acc + jnp.dot(x4, w_ref[3], preferred_element_type=_F32)
    o_ref[...] = acc + w0_ref[0:1, :]


def _kan1(hp, w, w0s):
    return pl.pallas_call(
        _kan1_body,
        grid=(NBLK,),
        in_specs=[
            pl.BlockSpec((BLK, HID), lambda i: (i, 0)),
            pl.BlockSpec((4, HID, HID), lambda i: (0, 0, 0)),
            pl.BlockSpec((8, HID), lambda i: (0, 0)),
        ],
        out_specs=pl.BlockSpec((BLK, HID), lambda i: (i, 0)),
        out_shape=jax.ShapeDtypeStruct((NP, HID), _F32),
    )(hp, w, w0s)


def _route_body(src_hbm, dst_hbm, rsrc_hbm, rdst_hbm, cnt_hbm,
                ins, ind, outs, outd, cntv):
    c = lax.axis_index("c")
    s = lax.axis_index("s")
    wid = c * NTILE + s
    lo = c * HALF

    pltpu.sync_copy(src_hbm.at[pl.ds(s * EPT, EPT)], ins)
    pltpu.sync_copy(dst_hbm.at[pl.ds(s * EPT, EPT)], ind)

    lanes = lax.iota(_I32, 16)
    dummy_src = s * 625 + lanes
    dummy_dst = HALF + s * 8 + lanes % 8

    @pl.loop(0, CAP, step=16)
    def _(i):
        outs[pl.ds(i, 16)] = dummy_src
        outd[pl.ds(i, 16)] = dummy_dst

    @pl.loop(0, EPT // 16, init_carry=jnp.int32(0))
    def pos(i, p):
        d = ind[pl.ds(i * 16, 16)]
        sr = ins[pl.ds(i * 16, 16)]
        m = (d >= lo) & (d < lo + HALF)
        mi = m.astype(_I32)
        slot = p + plsc.cumsum(mi) - mi
        plsc.store_scatter(outd, [slot], d - lo, mask=m)
        plsc.store_scatter(outs, [slot], sr, mask=m)
        return p + jnp.sum(mi)

    nch2 = 2 * lax.div(pos + 255, jnp.int32(256))
    cntv[...] = jnp.broadcast_to(nch2, (16,))

    pltpu.sync_copy(outs, rsrc_hbm.at[wid])
    pltpu.sync_copy(outd, rdst_hbm.at[wid])
    pltpu.sync_copy(cntv, cnt_hbm.at[wid])


def _route(ei0, ei1):
    mesh = plsc.VectorSubcoreMesh(core_axis_name="c", subcore_axis_name="s")
    kern = pl.kernel(
        _route_body,
        out_type=(
            jax.ShapeDtypeStruct((NSC * NTILE, CAP), _I32),
            jax.ShapeDtypeStruct((NSC * NTILE, CAP), _I32),
            jax.ShapeDtypeStruct((NSC * NTILE, 16), _I32),
        ),
        mesh=mesh,
        compiler_params=pltpu.CompilerParams(needs_layout_passes=False),
        scratch_types=[
            pltpu.VMEM((EPT,), _I32),
            pltpu.VMEM((EPT,), _I32),
            pltpu.VMEM((CAP,), _I32),
            pltpu.VMEM((CAP,), _I32),
            pltpu.VMEM((16,), _I32),
        ],
    )
    return kern(ei0, ei1)


def _agg_body(h_hbm, rsrc_hbm, rdst_hbm, cnt_hbm, zero_hbm, out_hbm,
              srcv, dstv, rows0, rows1, cntv, acc, semg0, semg1):
    c = lax.axis_index("c")
    s = lax.axis_index("s")
    wid = c * NTILE + s

    pltpu.sync_copy(zero_hbm, acc.at[pl.ds(s * RPT, RPT)])

    pltpu.sync_copy(rsrc_hbm.at[wid], srcv)
    pltpu.sync_copy(rdst_hbm.at[wid], dstv)
    pltpu.sync_copy(cnt_hbm.at[wid], cntv)
    nch = jnp.max(cntv[...])

    plsc.subcore_barrier()

    @pl.loop(0, nch, step=2)
    def _(i):
        g0 = pltpu.async_copy(h_hbm.at[srcv.at[i]], rows0, semg0)
        g1 = pltpu.async_copy(h_hbm.at[srcv.at[i + 1]], rows1, semg1)
        g0.wait()
        pltpu.sync_copy(rows0, acc.at[dstv.at[i]], add=True)
        g1.wait()
        pltpu.sync_copy(rows1, acc.at[dstv.at[i + 1]], add=True)

    plsc.subcore_barrier()

    @pl.when(s < NTILE - 1)
    def _():
        pltpu.sync_copy(acc.at[pl.ds(s * RPT, RPT)],
                        out_hbm.at[pl.ds(c * HALF + s * RPT, RPT)])

    @pl.when(s == NTILE - 1)
    def _():
        pltpu.sync_copy(acc.at[pl.ds((NTILE - 1) * RPT, HALF - (NTILE - 1) * RPT)],
                        out_hbm.at[pl.ds(c * HALF + (NTILE - 1) * RPT,
                                         HALF - (NTILE - 1) * RPT)])


def _agg(hp, rsrc3, rdst3, cnt, zeros):
    mesh = plsc.VectorSubcoreMesh(core_axis_name="c", subcore_axis_name="s")
    kern = pl.kernel(
        _agg_body,
        out_type=jax.ShapeDtypeStruct((NP, HID), _F32),
        mesh=mesh,
        compiler_params=pltpu.CompilerParams(needs_layout_passes=False),
        scratch_types=[
            pltpu.VMEM((CAPC, CH), _I32),
            pltpu.VMEM((CAPC, CH), _I32),
            pltpu.VMEM((CH, HID), _F32),
            pltpu.VMEM((CH, HID), _F32),
            pltpu.VMEM((16,), _I32),
            pltpu.VMEM_SHARED((ACC_R, HID), _F32),
            pltpu.SemaphoreType.DMA,
            pltpu.SemaphoreType.DMA,
        ],
    )
    return kern(hp, rsrc3, rdst3, cnt, zeros)


def _fourier_block(a_ref, y_ref, wc_ref, ws_ref):
    a = a_ref[...]
    c1 = jnp.cos(a)
    s1 = jnp.sin(a)
    c2 = c1 * c1 - s1 * s1
    s2 = 2.0 * c1 * s1
    c3 = c2 * c1 - s2 * s1
    s3 = s2 * c1 + c2 * s1
    c4 = c2 * c2 - s2 * s2
    s4 = 2.0 * s2 * c2
    acc = jnp.dot(c1, wc_ref[0], preferred_element_type=_F32)
    acc = acc + jnp.dot(c2, wc_ref[1], preferred_element_type=_F32)
    acc = acc + jnp.dot(c3, wc_ref[2], preferred_element_type=_F32)
    acc = acc + jnp.dot(c4, wc_ref[3], preferred_element_type=_F32)
    acc = acc + jnp.dot(s1, ws_ref[0], preferred_element_type=_F32)
    acc = acc + jnp.dot(s2, ws_ref[1], preferred_element_type=_F32)
    acc = acc + jnp.dot(s3, ws_ref[2], preferred_element_type=_F32)
    acc = acc + jnp.dot(s4, ws_ref[3], preferred_element_type=_F32)
    r = acc + y_ref[...]
    return jnp.where(r >= 0.0, r, 0.01 * r)


def _mid_body(a_ref, y_ref, wc_ref, ws_ref, o_ref):
    o_ref[...] = _fourier_block(a_ref, y_ref, wc_ref, ws_ref)


def _mid(p, y1, wc, ws):
    return pl.pallas_call(
        _mid_body,
        grid=(NBLK,),
        in_specs=[
            pl.BlockSpec((BLK, HID), lambda i: (i, 0)),
            pl.BlockSpec((BLK, HID), lambda i: (i, 0)),
            pl.BlockSpec((4, HID, HID), lambda i: (0, 0, 0)),
            pl.BlockSpec((4, HID, HID), lambda i: (0, 0, 0)),
        ],
        out_specs=pl.BlockSpec((BLK, HID), lambda i: (i, 0)),
        out_shape=jax.ShapeDtypeStruct((NP, HID), _F32),
    )(p, y1, wc, ws)


def _final_body(a_ref, y_ref, wc_ref, ws_ref, lwb_ref, o_ref, pool_ref):
    i = pl.program_id(0)

    @pl.when(i == 0)
    def _():
        pool_ref[...] = jnp.zeros_like(pool_ref)

    r = _fourier_block(a_ref, y_ref, wc_ref, ws_ref)
    rows = i * BLK + lax.broadcasted_iota(_I32, (BLK, HID), 0)
    r = jnp.where(rows < N, r, 0.0)
    pool_ref[0:1, :] = pool_ref[0:1, :] + jnp.sum(r, axis=0, keepdims=True)

    @pl.when(i == NBLK - 1)
    def _():
        pool = pool_ref[0:1, :]
        z = jnp.sum(lwb_ref[0:1, :] + pool * lwb_ref[1:2, :])
        o_ref[...] = jax.nn.sigmoid(z).reshape(1, 1)


def _final(p, h2, wc, ws, lwb):
    return pl.pallas_call(
        _final_body,
        grid=(NBLK,),
        in_specs=[
            pl.BlockSpec((BLK, HID), lambda i: (i, 0)),
            pl.BlockSpec((BLK, HID), lambda i: (i, 0)),
            pl.BlockSpec((4, HID, HID), lambda i: (0, 0, 0)),
            pl.BlockSpec((4, HID, HID), lambda i: (0, 0, 0)),
            pl.BlockSpec((8, HID), lambda i: (0, 0)),
        ],
        out_specs=pl.BlockSpec((1, 1), lambda i: (0, 0)),
        out_shape=jax.ShapeDtypeStruct((1, 1), _F32),
        scratch_shapes=[pltpu.VMEM((8, HID), _F32)],
    )(p, h2, wc, ws, lwb)


def kernel(h, edge_index, coeffs1, fc1, fc2, lin_coeffs, lin_bias):
    hp = jnp.pad(h, ((0, NP - N), (0, 0)))
    ei0 = edge_index[0]
    ei1 = edge_index[1]
    zeros = jnp.zeros((RPT, HID), _F32)

    w = jnp.stack([coeffs1[:, :, d].T for d in (1, 2, 3, 4)])
    w0s = jnp.broadcast_to(jnp.sum(coeffs1[:, :, 0], axis=1)[None, :],
                           (8, HID))
    wc1 = jnp.transpose(fc1[0], (2, 1, 0))
    ws1 = jnp.transpose(fc1[1], (2, 1, 0))
    wc2 = jnp.transpose(fc2[0], (2, 1, 0))
    ws2 = jnp.transpose(fc2[1], (2, 1, 0))
    lw0 = lin_coeffs[0, :, 0] + lin_bias[0, 0] / HID
    lw1 = lin_coeffs[0, :, 1]
    lwb = jnp.concatenate(
        [lw0[None, :], lw1[None, :], jnp.zeros((6, HID), _F32)], axis=0)

    rsrc, rdst, cnt = _route(ei0, ei1)
    rsrc3 = rsrc.reshape(NSC * NTILE, CAPC, CH)
    rdst3 = rdst.reshape(NSC * NTILE, CAPC, CH)

    y1 = _kan1(hp, w, w0s)
    p1 = _agg(y1, rsrc3, rdst3, cnt, zeros)
    h2 = _mid(p1, y1, wc1, ws1)
    p2 = _agg(h2, rsrc3, rdst3, cnt, zeros)
    return _final(p2, h2, wc2, ws2, lwb)


# speedup vs baseline: 12.1608x; 1.1999x over previous
import jax
import jax.numpy as jnp
from jax import lax
from jax.experimental import pallas as pl
from jax.experimental.pallas import tpu as pltpu
from jax.experimental.pallas import tpu_sc as plsc

N = 10000
NP = 10240
E = 320000
HID = 128
BLK = 512
NBLK = NP // BLK

NSC = 2
NTILE = 16
EPT = E // NTILE
HALF = NP // 2
TRASH = 128
ACC_R = HALF + TRASH
RPT = ACC_R // NTILE
CH = 128
CAPC = 160
CAP = CAPC * CH
NBUF = 4

_F32 = jnp.float32
_I32 = jnp.int32


def _kan1_body(x_ref, w_ref, w0_ref, o_ref):
    x = x_ref[...]
    x2 = x * x
    x3 = x2 * x
    x4 = x2 * x2
    acc = jnp.dot(x, w_ref[0], preferred_element_type=_F32)
    acc = acc + jnp.dot(x2, w_ref[1], preferred_element_type=_F32)
    acc = acc + jnp.dot(x3, w_ref[2], preferred_element_type=_F32)
    acc = acc + jnp.dot(x4, w_ref[3], preferred_element_type=_F32)
    o_ref[...] = acc + w0_ref[0:1, :]


def _kan1(hp, w, w0s):
    return pl.pallas_call(
        _kan1_body,
        grid=(NBLK,),
        in_specs=[
            pl.BlockSpec((BLK, HID), lambda i: (i, 0)),
            pl.BlockSpec((4, HID, HID), lambda i: (0, 0, 0)),
            pl.BlockSpec((8, HID), lambda i: (0, 0)),
        ],
        out_specs=pl.BlockSpec((BLK, HID), lambda i: (i, 0)),
        out_shape=jax.ShapeDtypeStruct((NP, HID), _F32),
    )(hp, w, w0s)


def _route_body(src_hbm, dst_hbm, rpk_hbm, cnt_hbm, ins, ind, outp, cntv):
    c = lax.axis_index("c")
    s = lax.axis_index("s")
    wid = c * NTILE + s
    lo = c * HALF

    pltpu.sync_copy(src_hbm.at[pl.ds(s * EPT, EPT)], ins)
    pltpu.sync_copy(dst_hbm.at[pl.ds(s * EPT, EPT)], ind)

    lanes = lax.iota(_I32, 16)
    dummy = (s * 625 + lanes) + ((HALF + s * 8 + lanes % 8) << 14)

    @pl.loop(0, CAP, step=16)
    def _(i):
        outp[pl.ds(i, 16)] = dummy

    @pl.loop(0, EPT // 16, init_carry=jnp.int32(0))
    def pos(i, p):
        d = ind[pl.ds(i * 16, 16)]
        sr = ins[pl.ds(i * 16, 16)]
        m = (d >= lo) & (d < lo + HALF)
        mi = m.astype(_I32)
        slot = p + plsc.cumsum(mi) - mi
        plsc.store_scatter(outp, [slot], sr + ((d - lo) << 14), mask=m)
        return p + jnp.sum(mi)

    nch4 = 4 * lax.div(pos + 511, jnp.int32(512))
    cntv[...] = jnp.broadcast_to(nch4, (16,))

    pltpu.sync_copy(outp, rpk_hbm.at[wid])
    pltpu.sync_copy(cntv, cnt_hbm.at[wid])


def _route(ei0, ei1):
    mesh = plsc.VectorSubcoreMesh(core_axis_name="c", subcore_axis_name="s")
    kern = pl.kernel(
        _route_body,
        out_type=(
            jax.ShapeDtypeStruct((NSC * NTILE, CAP), _I32),
            jax.ShapeDtypeStruct((NSC * NTILE, 16), _I32),
        ),
        mesh=mesh,
        compiler_params=pltpu.CompilerParams(needs_layout_passes=False),
        scratch_types=[
            pltpu.VMEM((EPT,), _I32),
            pltpu.VMEM((EPT,), _I32),
            pltpu.VMEM((CAP,), _I32),
            pltpu.VMEM((16,), _I32),
        ],
    )
    return kern(ei0, ei1)


def _agg_body(h_hbm, rpk_hbm, cnt_hbm, zero_hbm, out_hbm,
              pkv, srcv, dstv, rows0, rows1, rows2, rows3, cntv, acc,
              semg0, semg1, semg2, semg3, sems0, sems1, sems2, sems3):
    c = lax.axis_index("c")
    s = lax.axis_index("s")
    wid = c * NTILE + s
    bufs = (rows0, rows1, rows2, rows3)
    gsems = (semg0, semg1, semg2, semg3)
    ssems = (sems0, sems1, sems2, sems3)

    pltpu.sync_copy(zero_hbm, acc.at[pl.ds(s * RPT, RPT)])

    pltpu.sync_copy(rpk_hbm.at[wid], pkv)
    pltpu.sync_copy(cnt_hbm.at[wid], cntv)
    nch = jnp.max(cntv[...])

    plsc.subcore_barrier()

    def unpack(jc, k):
        for m in range(CH // 16):
            v = pkv[jc, pl.ds(m * 16, 16)]
            srcv[k, pl.ds(m * 16, 16)] = v & 0x3FFF
            dstv[k, pl.ds(m * 16, 16)] = v >> 14

    for k in range(NBUF):
        unpack(jnp.int32(k), k)
        pltpu.async_copy(h_hbm.at[srcv.at[k]], bufs[k], gsems[k])

    @pl.loop(0, nch, step=NBUF)
    def _(j):
        for k in range(NBUF):
            pltpu.make_async_copy(h_hbm.at[srcv.at[0]], bufs[k],
                                  gsems[k]).wait()
            pltpu.async_copy(bufs[k], acc.at[dstv.at[k]], ssems[k],
                             add=True)
        for k in range(NBUF):
            pltpu.make_async_copy(bufs[k], acc.at[dstv.at[0]],
                                  ssems[k]).wait()
            jc = jnp.minimum(j + NBUF + k, CAPC - 1)
            unpack(jc, k)
            pltpu.async_copy(h_hbm.at[srcv.at[k]], bufs[k], gsems[k])

    for k in range(NBUF):
        pltpu.make_async_copy(h_hbm.at[srcv.at[0]], bufs[k], gsems[k]).wait()

    plsc.subcore_barrier()

    @pl.when(s < NTILE - 1)
    def _():
        pltpu.sync_copy(acc.at[pl.ds(s * RPT, RPT)],
                        out_hbm.at[pl.ds(c * HALF + s * RPT, RPT)])

    @pl.when(s == NTILE - 1)
    def _():
        pltpu.sync_copy(acc.at[pl.ds((NTILE - 1) * RPT, HALF - (NTILE - 1) * RPT)],
                        out_hbm.at[pl.ds(c * HALF + (NTILE - 1) * RPT,
                                         HALF - (NTILE - 1) * RPT)])


def _agg(hp, rpk3, cnt, zeros):
    mesh = plsc.VectorSubcoreMesh(core_axis_name="c", subcore_axis_name="s")
    kern = pl.kernel(
        _agg_body,
        out_type=jax.ShapeDtypeStruct((NP, HID), _F32),
        mesh=mesh,
        compiler_params=pltpu.CompilerParams(needs_layout_passes=False),
        scratch_types=[
            pltpu.VMEM((CAPC, CH), _I32),
            pltpu.VMEM((NBUF, CH), _I32),
            pltpu.VMEM((NBUF, CH), _I32),
            pltpu.VMEM((CH, HID), _F32),
            pltpu.VMEM((CH, HID), _F32),
            pltpu.VMEM((CH, HID), _F32),
            pltpu.VMEM((CH, HID), _F32),
            pltpu.VMEM((16,), _I32),
            pltpu.VMEM_SHARED((ACC_R, HID), _F32),
            pltpu.SemaphoreType.DMA,
            pltpu.SemaphoreType.DMA,
            pltpu.SemaphoreType.DMA,
            pltpu.SemaphoreType.DMA,
            pltpu.SemaphoreType.DMA,
            pltpu.SemaphoreType.DMA,
            pltpu.SemaphoreType.DMA,
            pltpu.SemaphoreType.DMA,
        ],
    )
    return kern(hp, rpk3, cnt, zeros)


def _fourier_block(a_ref, y_ref, wc_ref, ws_ref):
    a = a_ref[...]
    c1 = jnp.cos(a)
    s1 = jnp.sin(a)
    c2 = c1 * c1 - s1 * s1
    s2 = 2.0 * c1 * s1
    c3 = c2 * c1 - s2 * s1
    s3 = s2 * c1 + c2 * s1
    c4 = c2 * c2 - s2 * s2
    s4 = 2.0 * s2 * c2
    acc = jnp.dot(c1, wc_ref[0], preferred_element_type=_F32)
    acc = acc + jnp.dot(c2, wc_ref[1], preferred_element_type=_F32)
    acc = acc + jnp.dot(c3, wc_ref[2], preferred_element_type=_F32)
    acc = acc + jnp.dot(c4, wc_ref[3], preferred_element_type=_F32)
    acc = acc + jnp.dot(s1, ws_ref[0], preferred_element_type=_F32)
    acc = acc + jnp.dot(s2, ws_ref[1], preferred_element_type=_F32)
    acc = acc + jnp.dot(s3, ws_ref[2], preferred_element_type=_F32)
    acc = acc + jnp.dot(s4, ws_ref[3], preferred_element_type=_F32)
    r = acc + y_ref[...]
    return jnp.where(r >= 0.0, r, 0.01 * r)


def _mid_body(a_ref, y_ref, wc_ref, ws_ref, o_ref):
    o_ref[...] = _fourier_block(a_ref, y_ref, wc_ref, ws_ref)


def _mid(p, y1, wc, ws):
    return pl.pallas_call(
        _mid_body,
        grid=(NBLK,),
        in_specs=[
            pl.BlockSpec((BLK, HID), lambda i: (i, 0)),
            pl.BlockSpec((BLK, HID), lambda i: (i, 0)),
            pl.BlockSpec((4, HID, HID), lambda i: (0, 0, 0)),
            pl.BlockSpec((4, HID, HID), lambda i: (0, 0, 0)),
        ],
        out_specs=pl.BlockSpec((BLK, HID), lambda i: (i, 0)),
        out_shape=jax.ShapeDtypeStruct((NP, HID), _F32),
    )(p, y1, wc, ws)


def _final_body(a_ref, y_ref, wc_ref, ws_ref, lwb_ref, o_ref, pool_ref):
    i = pl.program_id(0)

    @pl.when(i == 0)
    def _():
        pool_ref[...] = jnp.zeros_like(pool_ref)

    r = _fourier_block(a_ref, y_ref, wc_ref, ws_ref)
    rows = i * BLK + lax.broadcasted_iota(_I32, (BLK, HID), 0)
    r = jnp.where(rows < N, r, 0.0)
    pool_ref[0:1, :] = pool_ref[0:1, :] + jnp.sum(r, axis=0, keepdims=True)

    @pl.when(i == NBLK - 1)
    def _():
        pool = pool_ref[0:1, :]
        z = jnp.sum(lwb_ref[0:1, :] + pool * lwb_ref[1:2, :])
        o_ref[...] = jax.nn.sigmoid(z).reshape(1, 1)


def _final(p, h2, wc, ws, lwb):
    return pl.pallas_call(
        _final_body,
        grid=(NBLK,),
        in_specs=[
            pl.BlockSpec((BLK, HID), lambda i: (i, 0)),
            pl.BlockSpec((BLK, HID), lambda i: (i, 0)),
            pl.BlockSpec((4, HID, HID), lambda i: (0, 0, 0)),
            pl.BlockSpec((4, HID, HID), lambda i: (0, 0, 0)),
            pl.BlockSpec((8, HID), lambda i: (0, 0)),
        ],
        out_specs=pl.BlockSpec((1, 1), lambda i: (0, 0)),
        out_shape=jax.ShapeDtypeStruct((1, 1), _F32),
        scratch_shapes=[pltpu.VMEM((8, HID), _F32)],
    )(p, h2, wc, ws, lwb)


def kernel(h, edge_index, coeffs1, fc1, fc2, lin_coeffs, lin_bias):
    hp = jnp.pad(h, ((0, NP - N), (0, 0)))
    ei0 = edge_index[0]
    ei1 = edge_index[1]
    zeros = jnp.zeros((RPT, HID), _F32)

    w = jnp.stack([coeffs1[:, :, d].T for d in (1, 2, 3, 4)])
    w0s = jnp.broadcast_to(jnp.sum(coeffs1[:, :, 0], axis=1)[None, :],
                           (8, HID))
    wc1 = jnp.transpose(fc1[0], (2, 1, 0))
    ws1 = jnp.transpose(fc1[1], (2, 1, 0))
    wc2 = jnp.transpose(fc2[0], (2, 1, 0))
    ws2 = jnp.transpose(fc2[1], (2, 1, 0))
    lw0 = lin_coeffs[0, :, 0] + lin_bias[0, 0] / HID
    lw1 = lin_coeffs[0, :, 1]
    lwb = jnp.concatenate(
        [lw0[None, :], lw1[None, :], jnp.zeros((6, HID), _F32)], axis=0)

    rpk, cnt = _route(ei0, ei1)
    rpk3 = rpk.reshape(NSC * NTILE, CAPC, CH)

    y1 = _kan1(hp, w, w0s)
    p1 = _agg(y1, rpk3, cnt, zeros)
    h2 = _mid(p1, y1, wc1, ws1)
    p2 = _agg(h2, rpk3, cnt, zeros)
    return _final(p2, h2, wc2, ws2, lwb)


# speedup vs baseline: 12.2216x; 1.0050x over previous
import jax
import jax.numpy as jnp
from jax import lax
from jax.experimental import pallas as pl
from jax.experimental.pallas import tpu as pltpu
from jax.experimental.pallas import tpu_sc as plsc

N = 10000
NP = 10240
E = 320000
HID = 128
BLK = 512
NBLK = NP // BLK

NSC = 2
NTILE = 16
EPT = E // NTILE
HALF = NP // 2
TRASH = 128
ACC_R = HALF + TRASH
RPT = ACC_R // NTILE
CH = 128
CAPC = 160
CAP = CAPC * CH
NBUF = 4

_F32 = jnp.float32
_I32 = jnp.int32


def _dot(a, b):
    return jax.lax.dot(a, b, preferred_element_type=_F32)


def _kan1_body(x_ref, w_ref, w0_ref, o_ref):
    x = x_ref[...]
    x2 = x * x
    x3 = x2 * x
    x4 = x2 * x2
    acc = _dot(x, w_ref[0])
    acc = acc + _dot(x2, w_ref[1])
    acc = acc + _dot(x3, w_ref[2])
    acc = acc + _dot(x4, w_ref[3])
    o_ref[...] = acc + w0_ref[0:1, :]


def _kan1(hp, w, w0s):
    return pl.pallas_call(
        _kan1_body,
        grid=(NBLK,),
        in_specs=[
            pl.BlockSpec((BLK, HID), lambda i: (i, 0)),
            pl.BlockSpec((4, HID, HID), lambda i: (0, 0, 0)),
            pl.BlockSpec((8, HID), lambda i: (0, 0)),
        ],
        out_specs=pl.BlockSpec((BLK, HID), lambda i: (i, 0)),
        out_shape=jax.ShapeDtypeStruct((N, HID), _F32),
    )(hp, w, w0s)


def _route_body(src_hbm, dst_hbm, rpk_hbm, cnt_hbm, ins, ind, outp, cntv):
    c = lax.axis_index("c")
    s = lax.axis_index("s")
    wid = c * NTILE + s
    lo = c * HALF

    pltpu.sync_copy(src_hbm.at[pl.ds(s * EPT, EPT)], ins)
    pltpu.sync_copy(dst_hbm.at[pl.ds(s * EPT, EPT)], ind)

    lanes = lax.iota(_I32, 16)
    dummy = (s * 625 + lanes) + ((HALF + s * 8 + lanes % 8) << 14)

    @pl.loop(0, CAPC)
    def _(r):
        for m in range(CH // 16):
            outp[r, pl.ds(m * 16, 16)] = dummy

    @pl.loop(0, EPT // 16, init_carry=jnp.int32(0))
    def pos(i, p):
        d = ind[pl.ds(i * 16, 16)]
        sr = ins[pl.ds(i * 16, 16)]
        m = (d >= lo) & (d < lo + HALF)
        mi = m.astype(_I32)
        slot = p + plsc.cumsum(mi) - mi
        plsc.store_scatter(outp, [slot >> 7, slot & 127],
                           sr + ((d - lo) << 14), mask=m)
        return p + jnp.sum(mi)

    nch4 = 4 * lax.div(pos + 511, jnp.int32(512))
    cntv[...] = jnp.broadcast_to(nch4, (16,))

    pltpu.sync_copy(outp, rpk_hbm.at[wid])
    pltpu.sync_copy(cntv, cnt_hbm.at[wid])


def _route(ei0, ei1):
    mesh = plsc.VectorSubcoreMesh(core_axis_name="c", subcore_axis_name="s")
    kern = pl.kernel(
        _route_body,
        out_type=(
            jax.ShapeDtypeStruct((NSC * NTILE, CAPC, CH), _I32),
            jax.ShapeDtypeStruct((NSC * NTILE, 16), _I32),
        ),
        mesh=mesh,
        compiler_params=pltpu.CompilerParams(needs_layout_passes=False),
        scratch_types=[
            pltpu.VMEM((EPT,), _I32),
            pltpu.VMEM((EPT,), _I32),
            pltpu.VMEM((CAPC, CH), _I32),
            pltpu.VMEM((16,), _I32),
        ],
    )
    return kern(ei0, ei1)


def _agg_body(h_hbm, rpk_hbm, cnt_hbm, zero_hbm, out_hbm,
              pkv, srcv, dstv, rows0, rows1, rows2, rows3, cntv, acc,
              semg0, semg1, semg2, semg3, sems0, sems1, sems2, sems3):
    c = lax.axis_index("c")
    s = lax.axis_index("s")
    wid = c * NTILE + s
    bufs = (rows0, rows1, rows2, rows3)
    gsems = (semg0, semg1, semg2, semg3)
    ssems = (sems0, sems1, sems2, sems3)

    pltpu.sync_copy(zero_hbm, acc.at[pl.ds(s * RPT, RPT)])

    pltpu.sync_copy(rpk_hbm.at[wid], pkv)
    pltpu.sync_copy(cnt_hbm.at[wid], cntv)
    nch = jnp.max(cntv[...])

    plsc.subcore_barrier()

    def unpack(jc, k):
        for m in range(CH // 16):
            v = pkv[jc, pl.ds(m * 16, 16)]
            srcv[k, pl.ds(m * 16, 16)] = v & 0x3FFF
            dstv[k, pl.ds(m * 16, 16)] = v >> 14

    for k in range(NBUF):
        unpack(jnp.int32(k), k)
        pltpu.async_copy(h_hbm.at[srcv.at[k]], bufs[k], gsems[k])

    @pl.loop(0, nch, step=NBUF)
    def _(j):
        for k in range(NBUF):
            pltpu.make_async_copy(h_hbm.at[srcv.at[0]], bufs[k],
                                  gsems[k]).wait()
            pltpu.async_copy(bufs[k], acc.at[dstv.at[k]], ssems[k],
                             add=True)
        for k in range(NBUF):
            pltpu.make_async_copy(bufs[k], acc.at[dstv.at[0]],
                                  ssems[k]).wait()
            jc = jnp.minimum(j + NBUF + k, CAPC - 1)
            unpack(jc, k)
            pltpu.async_copy(h_hbm.at[srcv.at[k]], bufs[k], gsems[k])

    for k in range(NBUF):
        pltpu.make_async_copy(h_hbm.at[srcv.at[0]], bufs[k], gsems[k]).wait()

    plsc.subcore_barrier()

    @pl.when(s < NTILE - 1)
    def _():
        pltpu.sync_copy(acc.at[pl.ds(s * RPT, RPT)],
                        out_hbm.at[pl.ds(c * HALF + s * RPT, RPT)])

    @pl.when(s == NTILE - 1)
    def _():
        pltpu.sync_copy(acc.at[pl.ds((NTILE - 1) * RPT, HALF - (NTILE - 1) * RPT)],
                        out_hbm.at[pl.ds(c * HALF + (NTILE - 1) * RPT,
                                         HALF - (NTILE - 1) * RPT)])


def _agg(hp, rpk3, cnt, zeros):
    mesh = plsc.VectorSubcoreMesh(core_axis_name="c", subcore_axis_name="s")
    kern = pl.kernel(
        _agg_body,
        out_type=jax.ShapeDtypeStruct((NP, HID), _F32),
        mesh=mesh,
        compiler_params=pltpu.CompilerParams(needs_layout_passes=False),
        scratch_types=[
            pltpu.VMEM((CAPC, CH), _I32),
            pltpu.VMEM((NBUF, CH), _I32),
            pltpu.VMEM((NBUF, CH), _I32),
            pltpu.VMEM((CH, HID), _F32),
            pltpu.VMEM((CH, HID), _F32),
            pltpu.VMEM((CH, HID), _F32),
            pltpu.VMEM((CH, HID), _F32),
            pltpu.VMEM((16,), _I32),
            pltpu.VMEM_SHARED((ACC_R, HID), _F32),
            pltpu.SemaphoreType.DMA,
            pltpu.SemaphoreType.DMA,
            pltpu.SemaphoreType.DMA,
            pltpu.SemaphoreType.DMA,
            pltpu.SemaphoreType.DMA,
            pltpu.SemaphoreType.DMA,
            pltpu.SemaphoreType.DMA,
            pltpu.SemaphoreType.DMA,
        ],
    )
    return kern(hp, rpk3, cnt, zeros)


def _fourier_block(a_ref, y_ref, wc_ref, ws_ref):
    a = a_ref[...]
    c1 = jnp.cos(a)
    s1 = jnp.sin(a)
    c2 = c1 * c1 - s1 * s1
    s2 = 2.0 * c1 * s1
    c3 = c2 * c1 - s2 * s1
    s3 = s2 * c1 + c2 * s1
    c4 = c2 * c2 - s2 * s2
    s4 = 2.0 * s2 * c2
    acc = _dot(c1, wc_ref[0])
    acc = acc + _dot(c2, wc_ref[1])
    acc = acc + _dot(c3, wc_ref[2])
    acc = acc + _dot(c4, wc_ref[3])
    acc = acc + _dot(s1, ws_ref[0])
    acc = acc + _dot(s2, ws_ref[1])
    acc = acc + _dot(s3, ws_ref[2])
    acc = acc + _dot(s4, ws_ref[3])
    r = acc + y_ref[...]
    return jnp.where(r >= 0.0, r, 0.01 * r)


def _mid_body(a_ref, y_ref, wc_ref, ws_ref, o_ref):
    o_ref[...] = _fourier_block(a_ref, y_ref, wc_ref, ws_ref)


def _mid(p, y1, wc, ws):
    return pl.pallas_call(
        _mid_body,
        grid=(NBLK,),
        in_specs=[
            pl.BlockSpec((BLK, HID), lambda i: (i, 0)),
            pl.BlockSpec((BLK, HID), lambda i: (i, 0)),
            pl.BlockSpec((4, HID, HID), lambda i: (0, 0, 0)),
            pl.BlockSpec((4, HID, HID), lambda i: (0, 0, 0)),
        ],
        out_specs=pl.BlockSpec((BLK, HID), lambda i: (i, 0)),
        out_shape=jax.ShapeDtypeStruct((N, HID), _F32),
    )(p, y1, wc, ws)


def _final_body(a_ref, y_ref, wc_ref, ws_ref, lwb_ref, o_ref, pool_ref):
    i = pl.program_id(0)

    @pl.when(i == 0)
    def _():
        pool_ref[...] = jnp.zeros_like(pool_ref)

    r = _fourier_block(a_ref, y_ref, wc_ref, ws_ref)
    rows = i * BLK + lax.broadcasted_iota(_I32, (BLK, HID), 0)
    r = jnp.where(rows < N, r, 0.0)
    pool_ref[0:1, :] = pool_ref[0:1, :] + jnp.sum(r, axis=0, keepdims=True)

    @pl.when(i == NBLK - 1)
    def _():
        pool = pool_ref[0:1, :]
        z = jnp.sum(lwb_ref[0:1, :] + pool * lwb_ref[1:2, :])
        o_ref[...] = jax.nn.sigmoid(z).reshape(1, 1)


def _final(p, h2, wc, ws, lwb):
    return pl.pallas_call(
        _final_body,
        grid=(NBLK,),
        in_specs=[
            pl.BlockSpec((BLK, HID), lambda i: (i, 0)),
            pl.BlockSpec((BLK, HID), lambda i: (i, 0)),
            pl.BlockSpec((4, HID, HID), lambda i: (0, 0, 0)),
            pl.BlockSpec((4, HID, HID), lambda i: (0, 0, 0)),
            pl.BlockSpec((8, HID), lambda i: (0, 0)),
        ],
        out_specs=pl.BlockSpec((1, 1), lambda i: (0, 0)),
        out_shape=jax.ShapeDtypeStruct((1, 1), _F32),
        scratch_shapes=[pltpu.VMEM((8, HID), _F32)],
    )(p, h2, wc, ws, lwb)


def kernel(h, edge_index, coeffs1, fc1, fc2, lin_coeffs, lin_bias):
    zeros = jnp.zeros((RPT, HID), _F32)

    w = jnp.stack([coeffs1[:, :, d].T for d in (1, 2, 3, 4)])
    w0s = jnp.broadcast_to(jnp.sum(coeffs1[:, :, 0], axis=1)[None, :],
                           (8, HID))
    wc1 = jnp.transpose(fc1[0], (2, 1, 0))
    ws1 = jnp.transpose(fc1[1], (2, 1, 0))
    wc2 = jnp.transpose(fc2[0], (2, 1, 0))
    ws2 = jnp.transpose(fc2[1], (2, 1, 0))
    lw0 = lin_coeffs[0, :, 0] + lin_bias[0, 0] / HID
    lw1 = lin_coeffs[0, :, 1]
    lwb = jnp.concatenate(
        [lw0[None, :], lw1[None, :], jnp.zeros((6, HID), _F32)], axis=0)

    rpk3, cnt = _route(edge_index[0], edge_index[1])

    y1 = _kan1(h, w, w0s)
    p1 = _agg(y1, rpk3, cnt, zeros)
    h2 = _mid(p1, y1, wc1, ws1)
    p2 = _agg(h2, rpk3, cnt, zeros)
    return _final(p2, h2, wc2, ws2, lwb)


# speedup vs baseline: 12.2339x; 1.0010x over previous
import jax
import jax.numpy as jnp
from jax import lax
from jax.experimental import pallas as pl
from jax.experimental.pallas import tpu as pltpu
from jax.experimental.pallas import tpu_sc as plsc

N = 10000
NP = 10240
E = 320000
HID = 128
BLK = 512
NBLK = NP // BLK

NSC = 2
NTILE = 16
EPT = E // NTILE
HALF = NP // 2
TRASH = 128
ACC_R = HALF + TRASH
RPT = ACC_R // NTILE
CH = 128
CAPC = 160
CAP = CAPC * CH
NBUF = 4

_F32 = jnp.float32
_I32 = jnp.int32


def _dot(a, b):
    return jax.lax.dot(a, b, preferred_element_type=_F32)


def _kan1_body(x_ref, w_ref, w0_ref, o_ref):
    x = x_ref[...]
    x2 = x * x
    x3 = x2 * x
    x4 = x2 * x2
    acc = _dot(x, w_ref[0])
    acc = acc + _dot(x2, w_ref[1])
    acc = acc + _dot(x3, w_ref[2])
    acc = acc + _dot(x4, w_ref[3])
    o_ref[...] = acc + w0_ref[0:1, :]


def _kan1(hp, w, w0s):
    return pl.pallas_call(
        _kan1_body,
        grid=(NBLK,),
        in_specs=[
            pl.BlockSpec((BLK, HID), lambda i: (i, 0)),
            pl.BlockSpec((4, HID, HID), lambda i: (0, 0, 0)),
            pl.BlockSpec((8, HID), lambda i: (0, 0)),
        ],
        out_specs=pl.BlockSpec((BLK, HID), lambda i: (i, 0)),
        out_shape=jax.ShapeDtypeStruct((N, HID), _F32),
    )(hp, w, w0s)


def _route_body(src_hbm, dst_hbm, rpk_hbm, cnt_hbm, ins, ind, outp, cntv):
    c = lax.axis_index("c")
    s = lax.axis_index("s")
    wid = c * NTILE + s
    lo = c * HALF

    pltpu.sync_copy(src_hbm.at[pl.ds(s * EPT, EPT)], ins)
    pltpu.sync_copy(dst_hbm.at[pl.ds(s * EPT, EPT)], ind)

    lanes = lax.iota(_I32, 16)
    dummy = (s * 625 + lanes) + ((HALF + s * 8 + lanes % 8) << 14)

    @pl.loop(0, CAPC)
    def _(r):
        for m in range(CH // 16):
            outp[r, pl.ds(m * 16, 16)] = dummy

    @pl.loop(0, EPT // 16, init_carry=jnp.int32(0))
    def pos(i, p):
        d = ind[pl.ds(i * 16, 16)]
        sr = ins[pl.ds(i * 16, 16)]
        m = (d >= lo) & (d < lo + HALF)
        mi = m.astype(_I32)
        slot = p + plsc.cumsum(mi) - mi
        plsc.store_scatter(outp, [slot >> 7, slot & 127],
                           sr + ((d - lo) << 14), mask=m)
        return p + jnp.sum(mi)

    nch4 = 4 * lax.div(pos + 511, jnp.int32(512))
    cntv[...] = jnp.broadcast_to(nch4, (16,))

    pltpu.sync_copy(outp, rpk_hbm.at[wid])
    pltpu.sync_copy(cntv, cnt_hbm.at[wid])


def _route(ei0, ei1):
    mesh = plsc.VectorSubcoreMesh(core_axis_name="c", subcore_axis_name="s")
    kern = pl.kernel(
        _route_body,
        out_type=(
            jax.ShapeDtypeStruct((NSC * NTILE, CAPC, CH), _I32),
            jax.ShapeDtypeStruct((NSC * NTILE, 16), _I32),
        ),
        mesh=mesh,
        compiler_params=pltpu.CompilerParams(needs_layout_passes=False),
        scratch_types=[
            pltpu.VMEM((EPT,), _I32),
            pltpu.VMEM((EPT,), _I32),
            pltpu.VMEM((CAPC, CH), _I32),
            pltpu.VMEM((16,), _I32),
        ],
    )
    return kern(ei0, ei1)


def _agg_body(h_hbm, rpk_hbm, cnt_hbm, zero_hbm, out_hbm,
              pkv, srcv, dstv, rows0, rows1, rows2, rows3, cntv, acc,
              semg0, semg1, semg2, semg3, sems0, sems1, sems2, sems3):
    c = lax.axis_index("c")
    s = lax.axis_index("s")
    wid = c * NTILE + s
    bufs = (rows0, rows1, rows2, rows3)
    gsems = (semg0, semg1, semg2, semg3)
    ssems = (sems0, sems1, sems2, sems3)

    pltpu.sync_copy(zero_hbm, acc.at[pl.ds(s * RPT, RPT)])

    pltpu.sync_copy(rpk_hbm.at[wid], pkv)
    pltpu.sync_copy(cnt_hbm.at[wid], cntv)
    nch = jnp.max(cntv[...])

    plsc.subcore_barrier()

    def unpack(jc, slot):
        for m in range(CH // 16):
            v = pkv[jc, pl.ds(m * 16, 16)]
            srcv[slot, pl.ds(m * 16, 16)] = v & 0x3FFF
            dstv[slot, pl.ds(m * 16, 16)] = v >> 14

    for k in range(NBUF):
        unpack(jnp.int32(k), jnp.int32(k))
        pltpu.async_copy(h_hbm.at[srcv.at[k]], bufs[k], gsems[k])

    @pl.loop(0, nch, step=NBUF)
    def _(j):
        b = lax.rem(lax.div(j, NBUF), jnp.int32(2)) * NBUF
        bn = NBUF - b
        for k in range(NBUF):
            pltpu.make_async_copy(h_hbm.at[srcv.at[0]], bufs[k],
                                  gsems[k]).wait()
            pltpu.async_copy(bufs[k], acc.at[dstv.at[b + k]], ssems[k],
                             add=True)
        for k in range(NBUF):
            jc = jnp.minimum(j + NBUF + k, CAPC - 1)
            unpack(jc, bn + k)
        for k in range(NBUF):
            pltpu.make_async_copy(bufs[k], acc.at[dstv.at[0]],
                                  ssems[k]).wait()
            pltpu.async_copy(h_hbm.at[srcv.at[bn + k]], bufs[k], gsems[k])

    for k in range(NBUF):
        pltpu.make_async_copy(h_hbm.at[srcv.at[0]], bufs[k], gsems[k]).wait()

    plsc.subcore_barrier()

    @pl.when(s < NTILE - 1)
    def _():
        pltpu.sync_copy(acc.at[pl.ds(s * RPT, RPT)],
                        out_hbm.at[pl.ds(c * HALF + s * RPT, RPT)])

    @pl.when(s == NTILE - 1)
    def _():
        pltpu.sync_copy(acc.at[pl.ds((NTILE - 1) * RPT, HALF - (NTILE - 1) * RPT)],
                        out_hbm.at[pl.ds(c * HALF + (NTILE - 1) * RPT,
                                         HALF - (NTILE - 1) * RPT)])


def _agg(hp, rpk3, cnt, zeros):
    mesh = plsc.VectorSubcoreMesh(core_axis_name="c", subcore_axis_name="s")
    kern = pl.kernel(
        _agg_body,
        out_type=jax.ShapeDtypeStruct((NP, HID), _F32),
        mesh=mesh,
        compiler_params=pltpu.CompilerParams(needs_layout_passes=False),
        scratch_types=[
            pltpu.VMEM((CAPC, CH), _I32),
            pltpu.VMEM((2 * NBUF, CH), _I32),
            pltpu.VMEM((2 * NBUF, CH), _I32),
            pltpu.VMEM((CH, HID), _F32),
            pltpu.VMEM((CH, HID), _F32),
            pltpu.VMEM((CH, HID), _F32),
            pltpu.VMEM((CH, HID), _F32),
            pltpu.VMEM((16,), _I32),
            pltpu.VMEM_SHARED((ACC_R, HID), _F32),
            pltpu.SemaphoreType.DMA,
            pltpu.SemaphoreType.DMA,
            pltpu.SemaphoreType.DMA,
            pltpu.SemaphoreType.DMA,
            pltpu.SemaphoreType.DMA,
            pltpu.SemaphoreType.DMA,
            pltpu.SemaphoreType.DMA,
            pltpu.SemaphoreType.DMA,
        ],
    )
    return kern(hp, rpk3, cnt, zeros)


def _fourier_block(a_ref, y_ref, wc_ref, ws_ref):
    a = a_ref[...]
    c1 = jnp.cos(a)
    s1 = jnp.sin(a)
    c2 = c1 * c1 - s1 * s1
    s2 = 2.0 * c1 * s1
    c3 = c2 * c1 - s2 * s1
    s3 = s2 * c1 + c2 * s1
    c4 = c2 * c2 - s2 * s2
    s4 = 2.0 * s2 * c2
    acc = _dot(c1, wc_ref[0])
    acc = acc + _dot(c2, wc_ref[1])
    acc = acc + _dot(c3, wc_ref[2])
    acc = acc + _dot(c4, wc_ref[3])
    acc = acc + _dot(s1, ws_ref[0])
    acc = acc + _dot(s2, ws_ref[1])
    acc = acc + _dot(s3, ws_ref[2])
    acc = acc + _dot(s4, ws_ref[3])
    r = acc + y_ref[...]
    return jnp.where(r >= 0.0, r, 0.01 * r)


def _mid_body(a_ref, y_ref, wc_ref, ws_ref, o_ref):
    o_ref[...] = _fourier_block(a_ref, y_ref, wc_ref, ws_ref)


def _mid(p, y1, wc, ws):
    return pl.pallas_call(
        _mid_body,
        grid=(NBLK,),
        in_specs=[
            pl.BlockSpec((BLK, HID), lambda i: (i, 0)),
            pl.BlockSpec((BLK, HID), lambda i: (i, 0)),
            pl.BlockSpec((4, HID, HID), lambda i: (0, 0, 0)),
            pl.BlockSpec((4, HID, HID), lambda i: (0, 0, 0)),
        ],
        out_specs=pl.BlockSpec((BLK, HID), lambda i: (i, 0)),
        out_shape=jax.ShapeDtypeStruct((N, HID), _F32),
    )(p, y1, wc, ws)


def _final_body(a_ref, y_ref, wc_ref, ws_ref, lwb_ref, o_ref, pool_ref):
    i = pl.program_id(0)

    @pl.when(i == 0)
    def _():
        pool_ref[...] = jnp.zeros_like(pool_ref)

    r = _fourier_block(a_ref, y_ref, wc_ref, ws_ref)
    rows = i * BLK + lax.broadcasted_iota(_I32, (BLK, HID), 0)
    r = jnp.where(rows < N, r, 0.0)
    pool_ref[0:1, :] = pool_ref[0:1, :] + jnp.sum(r, axis=0, keepdims=True)

    @pl.when(i == NBLK - 1)
    def _():
        pool = pool_ref[0:1, :]
        z = jnp.sum(lwb_ref[0:1, :] + pool * lwb_ref[1:2, :])
        o_ref[...] = jax.nn.sigmoid(z).reshape(1, 1)


def _final(p, h2, wc, ws, lwb):
    return pl.pallas_call(
        _final_body,
        grid=(NBLK,),
        in_specs=[
            pl.BlockSpec((BLK, HID), lambda i: (i, 0)),
            pl.BlockSpec((BLK, HID), lambda i: (i, 0)),
            pl.BlockSpec((4, HID, HID), lambda i: (0, 0, 0)),
            pl.BlockSpec((4, HID, HID), lambda i: (0, 0, 0)),
            pl.BlockSpec((8, HID), lambda i: (0, 0)),
        ],
        out_specs=pl.BlockSpec((1, 1), lambda i: (0, 0)),
        out_shape=jax.ShapeDtypeStruct((1, 1), _F32),
        scratch_shapes=[pltpu.VMEM((8, HID), _F32)],
    )(p, h2, wc, ws, lwb)


def kernel(h, edge_index, coeffs1, fc1, fc2, lin_coeffs, lin_bias):
    zeros = jnp.zeros((RPT, HID), _F32)

    w = jnp.stack([coeffs1[:, :, d].T for d in (1, 2, 3, 4)])
    w0s = jnp.broadcast_to(jnp.sum(coeffs1[:, :, 0], axis=1)[None, :],
                           (8, HID))
    wc1 = jnp.transpose(fc1[0], (2, 1, 0))
    ws1 = jnp.transpose(fc1[1], (2, 1, 0))
    wc2 = jnp.transpose(fc2[0], (2, 1, 0))
    ws2 = jnp.transpose(fc2[1], (2, 1, 0))
    lw0 = lin_coeffs[0, :, 0] + lin_bias[0, 0] / HID
    lw1 = lin_coeffs[0, :, 1]
    lwb = jnp.concatenate(
        [lw0[None, :], lw1[None, :], jnp.zeros((6, HID), _F32)], axis=0)

    rpk3, cnt = _route(edge_index[0], edge_index[1])

    y1 = _kan1(h, w, w0s)
    p1 = _agg(y1, rpk3, cnt, zeros)
    h2 = _mid(p1, y1, wc1, ws1)
    p2 = _agg(h2, rpk3, cnt, zeros)
    return _final(p2, h2, wc2, ws2, lwb)


# speedup vs baseline: 12.3819x; 1.0121x over previous
import jax
import jax.numpy as jnp
from jax import lax
from jax.experimental import pallas as pl
from jax.experimental.pallas import tpu as pltpu
from jax.experimental.pallas import tpu_sc as plsc

N = 10000
NP = 10240
E = 320000
HID = 128
BLK = 1024
NBLK = NP // BLK

NSC = 2
NTILE = 16
EPT = E // NTILE
HALF = NP // 2
TRASH = 128
ACC_R = HALF + TRASH
RPT = ACC_R // NTILE
CH = 128
CAPC = 160
CAP = CAPC * CH
NBUF = 4

_F32 = jnp.float32
_I32 = jnp.int32


def _dot(a, b):
    return jax.lax.dot(a, b, preferred_element_type=_F32)


def _kan1_body(x_ref, w_ref, w0_ref, o_ref):
    x = x_ref[...]
    x2 = x * x
    x3 = x2 * x
    x4 = x2 * x2
    acc = _dot(x, w_ref[0])
    acc = acc + _dot(x2, w_ref[1])
    acc = acc + _dot(x3, w_ref[2])
    acc = acc + _dot(x4, w_ref[3])
    o_ref[...] = acc + w0_ref[0:1, :]


def _kan1(hp, w, w0s):
    return pl.pallas_call(
        _kan1_body,
        grid=(NBLK,),
        in_specs=[
            pl.BlockSpec((BLK, HID), lambda i: (i, 0)),
            pl.BlockSpec((4, HID, HID), lambda i: (0, 0, 0)),
            pl.BlockSpec((8, HID), lambda i: (0, 0)),
        ],
        out_specs=pl.BlockSpec((BLK, HID), lambda i: (i, 0)),
        out_shape=jax.ShapeDtypeStruct((N, HID), _F32),
    )(hp, w, w0s)


def _route_body(src_hbm, dst_hbm, rpk_hbm, cnt_hbm, ins, ind, outp, cntv):
    c = lax.axis_index("c")
    s = lax.axis_index("s")
    wid = c * NTILE + s
    lo = c * HALF

    pltpu.sync_copy(src_hbm.at[pl.ds(s * EPT, EPT)], ins)
    pltpu.sync_copy(dst_hbm.at[pl.ds(s * EPT, EPT)], ind)

    lanes = lax.iota(_I32, 16)
    dummy = (s * 625 + lanes) + ((HALF + s * 8 + lanes % 8) << 14)

    @pl.loop(0, CAPC)
    def _(r):
        for m in range(CH // 16):
            outp[r, pl.ds(m * 16, 16)] = dummy

    @pl.loop(0, EPT // 16, init_carry=jnp.int32(0))
    def pos(i, p):
        d = ind[pl.ds(i * 16, 16)]
        sr = ins[pl.ds(i * 16, 16)]
        m = (d >= lo) & (d < lo + HALF)
        mi = m.astype(_I32)
        slot = p + plsc.cumsum(mi) - mi
        plsc.store_scatter(outp, [slot >> 7, slot & 127],
                           sr + ((d - lo) << 14), mask=m)
        return p + jnp.sum(mi)

    nch4 = 4 * lax.div(pos + 511, jnp.int32(512))
    cntv[...] = jnp.broadcast_to(nch4, (16,))

    pltpu.sync_copy(outp, rpk_hbm.at[wid])
    pltpu.sync_copy(cntv, cnt_hbm.at[wid])


def _route(ei0, ei1):
    mesh = plsc.VectorSubcoreMesh(core_axis_name="c", subcore_axis_name="s")
    kern = pl.kernel(
        _route_body,
        out_type=(
            jax.ShapeDtypeStruct((NSC * NTILE, CAPC, CH), _I32),
            jax.ShapeDtypeStruct((NSC * NTILE, 16), _I32),
        ),
        mesh=mesh,
        compiler_params=pltpu.CompilerParams(needs_layout_passes=False),
        scratch_types=[
            pltpu.VMEM((EPT,), _I32),
            pltpu.VMEM((EPT,), _I32),
            pltpu.VMEM((CAPC, CH), _I32),
            pltpu.VMEM((16,), _I32),
        ],
    )
    return kern(ei0, ei1)


def _agg_body(h_hbm, rpk_hbm, cnt_hbm, zero_hbm, out_hbm,
              pkv, srcv, dstv, rows0, rows1, rows2, rows3, cntv, acc,
              semg0, semg1, semg2, semg3, sems0, sems1, sems2, sems3):
    c = lax.axis_index("c")
    s = lax.axis_index("s")
    wid = c * NTILE + s
    bufs = (rows0, rows1, rows2, rows3)
    gsems = (semg0, semg1, semg2, semg3)
    ssems = (sems0, sems1, sems2, sems3)

    pltpu.sync_copy(zero_hbm, acc.at[pl.ds(s * RPT, RPT)])

    pltpu.sync_copy(rpk_hbm.at[wid], pkv)
    pltpu.sync_copy(cnt_hbm.at[wid], cntv)
    nch = jnp.max(cntv[...])

    plsc.subcore_barrier()

    def unpack(jc, slot):
        for m in range(CH // 16):
            v = pkv[jc, pl.ds(m * 16, 16)]
            srcv[slot, pl.ds(m * 16, 16)] = v & 0x3FFF
            dstv[slot, pl.ds(m * 16, 16)] = v >> 14

    for k in range(NBUF):
        unpack(jnp.int32(k), jnp.int32(k))
        pltpu.async_copy(h_hbm.at[srcv.at[k]], bufs[k], gsems[k])

    @pl.loop(0, nch, step=NBUF)
    def _(j):
        b = lax.rem(lax.div(j, NBUF), jnp.int32(2)) * NBUF
        bn = NBUF - b
        for k in range(NBUF):
            pltpu.make_async_copy(h_hbm.at[srcv.at[0]], bufs[k],
                                  gsems[k]).wait()
            pltpu.async_copy(bufs[k], acc.at[dstv.at[b + k]], ssems[k],
                             add=True)
        for k in range(NBUF):
            jc = jnp.minimum(j + NBUF + k, CAPC - 1)
            unpack(jc, bn + k)
        for k in range(NBUF):
            pltpu.make_async_copy(bufs[k], acc.at[dstv.at[0]],
                                  ssems[k]).wait()
            pltpu.async_copy(h_hbm.at[srcv.at[bn + k]], bufs[k], gsems[k])

    for k in range(NBUF):
        pltpu.make_async_copy(h_hbm.at[srcv.at[0]], bufs[k], gsems[k]).wait()

    plsc.subcore_barrier()

    @pl.when(s < NTILE - 1)
    def _():
        pltpu.sync_copy(acc.at[pl.ds(s * RPT, RPT)],
                        out_hbm.at[pl.ds(c * HALF + s * RPT, RPT)])

    @pl.when(s == NTILE - 1)
    def _():
        pltpu.sync_copy(acc.at[pl.ds((NTILE - 1) * RPT, HALF - (NTILE - 1) * RPT)],
                        out_hbm.at[pl.ds(c * HALF + (NTILE - 1) * RPT,
                                         HALF - (NTILE - 1) * RPT)])


def _agg(hp, rpk3, cnt, zeros):
    mesh = plsc.VectorSubcoreMesh(core_axis_name="c", subcore_axis_name="s")
    kern = pl.kernel(
        _agg_body,
        out_type=jax.ShapeDtypeStruct((NP, HID), _F32),
        mesh=mesh,
        compiler_params=pltpu.CompilerParams(needs_layout_passes=False),
        scratch_types=[
            pltpu.VMEM((CAPC, CH), _I32),
            pltpu.VMEM((2 * NBUF, CH), _I32),
            pltpu.VMEM((2 * NBUF, CH), _I32),
            pltpu.VMEM((CH, HID), _F32),
            pltpu.VMEM((CH, HID), _F32),
            pltpu.VMEM((CH, HID), _F32),
            pltpu.VMEM((CH, HID), _F32),
            pltpu.VMEM((16,), _I32),
            pltpu.VMEM_SHARED((ACC_R, HID), _F32),
            pltpu.SemaphoreType.DMA,
            pltpu.SemaphoreType.DMA,
            pltpu.SemaphoreType.DMA,
            pltpu.SemaphoreType.DMA,
            pltpu.SemaphoreType.DMA,
            pltpu.SemaphoreType.DMA,
            pltpu.SemaphoreType.DMA,
            pltpu.SemaphoreType.DMA,
        ],
    )
    return kern(hp, rpk3, cnt, zeros)


def _fourier_block(a_ref, y_ref, wc_ref, ws_ref):
    a = a_ref[...]
    c1 = jnp.cos(a)
    s1 = jnp.sin(a)
    c2 = c1 * c1 - s1 * s1
    s2 = 2.0 * c1 * s1
    c3 = c2 * c1 - s2 * s1
    s3 = s2 * c1 + c2 * s1
    c4 = c2 * c2 - s2 * s2
    s4 = 2.0 * s2 * c2
    acc = _dot(c1, wc_ref[0])
    acc = acc + _dot(c2, wc_ref[1])
    acc = acc + _dot(c3, wc_ref[2])
    acc = acc + _dot(c4, wc_ref[3])
    acc = acc + _dot(s1, ws_ref[0])
    acc = acc + _dot(s2, ws_ref[1])
    acc = acc + _dot(s3, ws_ref[2])
    acc = acc + _dot(s4, ws_ref[3])
    r = acc + y_ref[...]
    return jnp.where(r >= 0.0, r, 0.01 * r)


def _mid_body(a_ref, y_ref, wc_ref, ws_ref, o_ref):
    o_ref[...] = _fourier_block(a_ref, y_ref, wc_ref, ws_ref)


def _mid(p, y1, wc, ws):
    return pl.pallas_call(
        _mid_body,
        grid=(NBLK,),
        in_specs=[
            pl.BlockSpec((BLK, HID), lambda i: (i, 0)),
            pl.BlockSpec((BLK, HID), lambda i: (i, 0)),
            pl.BlockSpec((4, HID, HID), lambda i: (0, 0, 0)),
            pl.BlockSpec((4, HID, HID), lambda i: (0, 0, 0)),
        ],
        out_specs=pl.BlockSpec((BLK, HID), lambda i: (i, 0)),
        out_shape=jax.ShapeDtypeStruct((N, HID), _F32),
    )(p, y1, wc, ws)


def _final_body(a_ref, y_ref, wc_ref, ws_ref, lwb_ref, o_ref, pool_ref):
    i = pl.program_id(0)

    @pl.when(i == 0)
    def _():
        pool_ref[...] = jnp.zeros_like(pool_ref)

    r = _fourier_block(a_ref, y_ref, wc_ref, ws_ref)
    rows = i * BLK + lax.broadcasted_iota(_I32, (BLK, HID), 0)
    r = jnp.where(rows < N, r, 0.0)
    pool_ref[0:1, :] = pool_ref[0:1, :] + jnp.sum(r, axis=0, keepdims=True)

    @pl.when(i == NBLK - 1)
    def _():
        pool = pool_ref[0:1, :]
        z = jnp.sum(lwb_ref[0:1, :] + pool * lwb_ref[1:2, :])
        o_ref[...] = jax.nn.sigmoid(z).reshape(1, 1)


def _final(p, h2, wc, ws, lwb):
    return pl.pallas_call(
        _final_body,
        grid=(NBLK,),
        in_specs=[
            pl.BlockSpec((BLK, HID), lambda i: (i, 0)),
            pl.BlockSpec((BLK, HID), lambda i: (i, 0)),
            pl.BlockSpec((4, HID, HID), lambda i: (0, 0, 0)),
            pl.BlockSpec((4, HID, HID), lambda i: (0, 0, 0)),
            pl.BlockSpec((8, HID), lambda i: (0, 0)),
        ],
        out_specs=pl.BlockSpec((1, 1), lambda i: (0, 0)),
        out_shape=jax.ShapeDtypeStruct((1, 1), _F32),
        scratch_shapes=[pltpu.VMEM((8, HID), _F32)],
    )(p, h2, wc, ws, lwb)


def kernel(h, edge_index, coeffs1, fc1, fc2, lin_coeffs, lin_bias):
    zeros = jnp.zeros((RPT, HID), _F32)

    w = jnp.stack([coeffs1[:, :, d].T for d in (1, 2, 3, 4)])
    w0s = jnp.broadcast_to(jnp.sum(coeffs1[:, :, 0], axis=1)[None, :],
                           (8, HID))
    wc1 = jnp.transpose(fc1[0], (2, 1, 0))
    ws1 = jnp.transpose(fc1[1], (2, 1, 0))
    wc2 = jnp.transpose(fc2[0], (2, 1, 0))
    ws2 = jnp.transpose(fc2[1], (2, 1, 0))
    lw0 = lin_coeffs[0, :, 0] + lin_bias[0, 0] / HID
    lw1 = lin_coeffs[0, :, 1]
    lwb = jnp.concatenate(
        [lw0[None, :], lw1[None, :], jnp.zeros((6, HID), _F32)], axis=0)

    rpk3, cnt = _route(edge_index[0], edge_index[1])

    y1 = _kan1(h, w, w0s)
    p1 = _agg(y1, rpk3, cnt, zeros)
    h2 = _mid(p1, y1, wc1, ws1)
    p2 = _agg(h2, rpk3, cnt, zeros)
    return _final(p2, h2, wc2, ws2, lwb)
